# baseline (speedup 1.0000x reference)
import functools

import jax
import jax.numpy as jnp
from jax.experimental import pallas as pl
from jax.experimental.pallas import tpu as pltpu

_B = 2
_N_SURF = 3000
_N_NM = 1000
_N = _N_SURF + _N_NM
_NP = 4096
_HEADS = 4
_HID = 64
_FEAT = _HEADS * _HID
_OUT_CH = 2
_DBLK = 256
_SBLK = 256
_NBLK = _NP // _DBLK
_RADIUS = 0.05
_PAD_VAL = 100.0


def _proj_body(x_ref, w_ref, amat_ref, h_ref, as_ref, ad_ref):
    x = x_ref[0]
    w = w_ref[...]
    h = jnp.dot(x, w, preferred_element_type=jnp.float32)
    h_ref[0] = h
    aa = jnp.dot(h, amat_ref[...], preferred_element_type=jnp.float32)
    as_ref[0] = aa[:, :_HEADS]
    ad_ref[0] = aa[:, _HEADS:]


def _attn_body(posd_ref, post_ref, h_ref, ast_ref, ad_ref, b_ref,
               wfc_ref, bfc_ref, o_ref, m_ref, l_ref, acc_ref, *, fuse_fc):
    k = pl.program_id(2)
    nk = pl.num_programs(2)

    @pl.when(k == 0)
    def _init():
        m_ref[...] = jnp.full((_DBLK, _HEADS), -1e30, jnp.float32)
        l_ref[...] = jnp.zeros((_DBLK, _HEADS), jnp.float32)
        acc_ref[...] = jnp.zeros((_DBLK, _FEAT), jnp.float32)

    pos_d = posd_ref[0]
    pos_s = post_ref[0]
    xd = pos_d[:, 0:1]
    xs = pos_s[0:1, :]
    gap_a = jnp.min(xs) - jnp.max(xd)
    gap_b = jnp.min(xd) - jnp.max(xs)
    live = jnp.maximum(gap_a, gap_b) < (_RADIUS + 1e-3)

    @pl.when(live)
    def _compute():
        sq_d = jnp.sum(pos_d * pos_d, axis=1, keepdims=True)
        sq_s = jnp.sum(pos_s * pos_s, axis=0, keepdims=True)
        dots = jnp.dot(pos_d, pos_s, preferred_element_type=jnp.float32)
        d2 = sq_d + sq_s - 2.0 * dots
        dist = jnp.sqrt(jnp.maximum(d2, 0.0))
        mask = dist < _RADIUS

        for hd in range(_HEADS):
            e = ad_ref[0, :, hd:hd + 1] + ast_ref[0, hd:hd + 1, :]
            e = jnp.where(e >= 0, e, 0.2 * e)
            cmax = jnp.max(jnp.where(mask, e, -jnp.inf), axis=1,
                           keepdims=True)
            m_old = m_ref[:, hd:hd + 1]
            mn = jnp.maximum(m_old, cmax)
            scale = jnp.exp(m_old - mn)
            ex = jnp.where(mask, jnp.exp(e - mn), 0.0)
            l_ref[:, hd:hd + 1] = (l_ref[:, hd:hd + 1] * scale +
                                   jnp.sum(ex, axis=1, keepdims=True))
            cs = slice(hd * _HID, (hd + 1) * _HID)
            acc_ref[:, cs] = acc_ref[:, cs] * scale + jnp.dot(
                ex, h_ref[0, :, cs], preferred_element_type=jnp.float32)
            m_ref[:, hd:hd + 1] = mn

    @pl.when(k == nk - 1)
    def _fin():
        den_full = jnp.concatenate(
            [jnp.broadcast_to(l_ref[:, hd:hd + 1], (_DBLK, _HID))
             for hd in range(_HEADS)], axis=1)
        out = acc_ref[...] / (den_full + 1e-16)
        out = jnp.maximum(out + b_ref[...], 0.0)
        if fuse_fc:
            out = jnp.dot(out, wfc_ref[...],
                          preferred_element_type=jnp.float32) + bfc_ref[...]
        o_ref[0] = out


def _amat(a_src, a_dst):
    eye = jnp.eye(_HEADS, dtype=jnp.float32)
    ms = (a_src[:, :, None] * eye[:, None, :]).reshape(_FEAT, _HEADS)
    md = (a_dst[:, :, None] * eye[:, None, :]).reshape(_FEAT, _HEADS)
    return jnp.concatenate([ms, md], axis=1)


def _proj(x, w, amat):
    cin = x.shape[-1]
    return pl.pallas_call(
        _proj_body,
        grid=(_B, _NBLK),
        in_specs=[
            pl.BlockSpec((1, _DBLK, cin), lambda b, j: (b, j, 0)),
            pl.BlockSpec((cin, _FEAT), lambda b, j: (0, 0)),
            pl.BlockSpec((_FEAT, 2 * _HEADS), lambda b, j: (0, 0)),
        ],
        out_specs=[
            pl.BlockSpec((1, _DBLK, _FEAT), lambda b, j: (b, j, 0)),
            pl.BlockSpec((1, _DBLK, _HEADS), lambda b, j: (b, j, 0)),
            pl.BlockSpec((1, _DBLK, _HEADS), lambda b, j: (b, j, 0)),
        ],
        out_shape=[
            jax.ShapeDtypeStruct((_B, _NP, _FEAT), jnp.float32),
            jax.ShapeDtypeStruct((_B, _NP, _HEADS), jnp.float32),
            jax.ShapeDtypeStruct((_B, _NP, _HEADS), jnp.float32),
        ],
        compiler_params=pltpu.CompilerParams(
            dimension_semantics=("parallel", "parallel")),
    )(x, w, amat)


def _attn(pos_nd, pos_t, h, as_t, ad, bias, wfc, bfc, fuse_fc):
    outc = _OUT_CH if fuse_fc else _FEAT
    nk = _NP // _SBLK
    return pl.pallas_call(
        functools.partial(_attn_body, fuse_fc=fuse_fc),
        grid=(_B, _NBLK, nk),
        in_specs=[
            pl.BlockSpec((1, _DBLK, 3), lambda b, j, k: (b, j, 0)),
            pl.BlockSpec((1, 3, _SBLK), lambda b, j, k: (b, 0, k)),
            pl.BlockSpec((1, _SBLK, _FEAT), lambda b, j, k: (b, k, 0)),
            pl.BlockSpec((1, _HEADS, _SBLK), lambda b, j, k: (b, 0, k)),
            pl.BlockSpec((1, _DBLK, _HEADS), lambda b, j, k: (b, j, 0)),
            pl.BlockSpec((1, _FEAT), lambda b, j, k: (0, 0)),
            pl.BlockSpec((_FEAT, _OUT_CH), lambda b, j, k: (0, 0)),
            pl.BlockSpec((1, _OUT_CH), lambda b, j, k: (0, 0)),
        ],
        out_specs=pl.BlockSpec((1, _DBLK, outc), lambda b, j, k: (b, j, 0)),
        out_shape=jax.ShapeDtypeStruct((_B, _NP, outc), jnp.float32),
        scratch_shapes=[
            pltpu.VMEM((_DBLK, _HEADS), jnp.float32),
            pltpu.VMEM((_DBLK, _HEADS), jnp.float32),
            pltpu.VMEM((_DBLK, _FEAT), jnp.float32),
        ],
        compiler_params=pltpu.CompilerParams(
            dimension_semantics=("parallel", "parallel", "arbitrary")),
    )(pos_nd, pos_t, h, as_t, ad, bias, wfc, bfc)


def kernel(pos, pos_non_manifold, W1, a1_src, a1_dst, b1,
           W2, a2_src, a2_dst, b2, W_fc, b_fc):
    pos_t = jnp.concatenate([pos, pos_non_manifold], axis=2)
    perm = jnp.argsort(pos_t[:, 0, :], axis=1)
    inv = jnp.argsort(perm, axis=1)
    pos_t = jnp.take_along_axis(pos_t, perm[:, None, :], axis=2)
    pos_t = jnp.pad(pos_t, ((0, 0), (0, 0), (0, _NP - _N)),
                    constant_values=_PAD_VAL)
    pos_nd = pos_t.transpose(0, 2, 1)

    b1r = b1.reshape(1, _FEAT)
    b2r = b2.reshape(1, _FEAT)
    bfc = b_fc.reshape(1, _OUT_CH)

    h, as_, ad = _proj(pos_nd, W1, _amat(a1_src, a1_dst))
    x1 = _attn(pos_nd, pos_t, h, as_.transpose(0, 2, 1), ad, b1r,
               W_fc, bfc, fuse_fc=False)

    h2, as2, ad2 = _proj(x1, W2, _amat(a2_src, a2_dst))
    y = _attn(pos_nd, pos_t, h2, as2.transpose(0, 2, 1), ad2, b2r,
              W_fc, bfc, fuse_fc=True)

    y = jnp.take_along_axis(y[:, :_N], inv[:, :, None], axis=1)
    return y[:, _N_NM:_N].reshape(_B, _OUT_CH, _N_SURF)


# speedup vs baseline: 3.3952x; 3.3952x over previous
import functools

import jax
import jax.numpy as jnp
from jax.experimental import pallas as pl
from jax.experimental.pallas import tpu as pltpu

_B = 2
_N_SURF = 3000
_N_NM = 1000
_N = _N_SURF + _N_NM
_NP = 4096
_HEADS = 4
_HID = 64
_FEAT = _HEADS * _HID
_OUT_CH = 2
_PBLK = 512
_DBLK = 256
_SBLK = 256
_NDB = _NP // _DBLK
_NSB = _NP // _SBLK
_RADIUS = 0.05
_TH = _RADIUS + 1e-3
_PAD_VAL = 100.0


def _proj_body(x_ref, w_ref, amat_ref, h_ref, as_ref, ad_ref):
    x = x_ref[0]
    w = w_ref[...]
    h = jnp.dot(x, w, preferred_element_type=jnp.float32)
    h_ref[0] = h
    aa = jnp.dot(h, amat_ref[...], preferred_element_type=jnp.float32)
    as_ref[0] = aa[:, :_HEADS]
    ad_ref[0] = aa[:, _HEADS:]


def _attn_body(lo_ref, hi_ref, posdt_ref, posnd_ref, h_ref, as_ref, adt_ref,
               b_ref, wfc_ref, bfc_ref, o_ref, *, fuse_fc):
    b = pl.program_id(0)
    j = pl.program_id(1)
    pos_dt = posdt_ref[0]
    sq_d = jnp.sum(pos_dt * pos_dt, axis=0, keepdims=True)
    ad_row = adt_ref[0]
    lo = lo_ref[b, j]
    hi = hi_ref[b, j]

    def chunk(k, carry):
        ms, ls, accs = carry
        off = k * _SBLK
        pos_s = posnd_ref[0, pl.ds(off, _SBLK), :]
        sq_s = jnp.sum(pos_s * pos_s, axis=1, keepdims=True)
        dots = jnp.dot(pos_s, pos_dt, preferred_element_type=jnp.float32)
        d2 = sq_s + sq_d - 2.0 * dots
        mask = jnp.sqrt(jnp.maximum(d2, 0.0)) < _RADIUS
        h_chunk = h_ref[0, pl.ds(off, _SBLK), :]
        as_chunk = as_ref[0, pl.ds(off, _SBLK), :]
        nm, nl, na = [], [], []
        for hd in range(_HEADS):
            e = as_chunk[:, hd:hd + 1] + ad_row[hd:hd + 1, :]
            e = jnp.where(e >= 0, e, 0.2 * e)
            cmax = jnp.max(jnp.where(mask, e, -jnp.inf), axis=0,
                           keepdims=True)
            mn = jnp.maximum(ms[hd], cmax)
            scale = jnp.exp(ms[hd] - mn)
            ex = jnp.where(mask, jnp.exp(e - mn), 0.0)
            nl.append(ls[hd] * scale + jnp.sum(ex, axis=0, keepdims=True))
            hc = h_chunk[:, hd * _HID:(hd + 1) * _HID]
            contrib = jax.lax.dot_general(
                hc, ex, (((0,), (0,)), ((), ())),
                preferred_element_type=jnp.float32)
            na.append(accs[hd] * scale + contrib)
            nm.append(mn)
        return tuple(nm), tuple(nl), tuple(na)

    init = (
        tuple(jnp.full((1, _DBLK), -1e30, jnp.float32) for _ in range(_HEADS)),
        tuple(jnp.zeros((1, _DBLK), jnp.float32) for _ in range(_HEADS)),
        tuple(jnp.zeros((_HID, _DBLK), jnp.float32) for _ in range(_HEADS)),
    )
    ms, ls, accs = jax.lax.fori_loop(lo, hi + 1, chunk, init)

    out = jnp.concatenate(
        [jnp.transpose(accs[hd] / (ls[hd] + 1e-16)) for hd in range(_HEADS)],
        axis=1)
    out = jnp.maximum(out + b_ref[...], 0.0)
    if fuse_fc:
        out = jnp.dot(out, wfc_ref[...],
                      preferred_element_type=jnp.float32) + bfc_ref[...]
    o_ref[0] = out


def _amat(a_src, a_dst):
    eye = jnp.eye(_HEADS, dtype=jnp.float32)
    ms = (a_src[:, :, None] * eye[:, None, :]).reshape(_FEAT, _HEADS)
    md = (a_dst[:, :, None] * eye[:, None, :]).reshape(_FEAT, _HEADS)
    return jnp.concatenate([ms, md], axis=1)


def _proj(x, w, amat):
    cin = x.shape[-1]
    return pl.pallas_call(
        _proj_body,
        grid=(_B, _NP // _PBLK),
        in_specs=[
            pl.BlockSpec((1, _PBLK, cin), lambda b, j: (b, j, 0)),
            pl.BlockSpec((cin, _FEAT), lambda b, j: (0, 0)),
            pl.BlockSpec((_FEAT, 2 * _HEADS), lambda b, j: (0, 0)),
        ],
        out_specs=[
            pl.BlockSpec((1, _PBLK, _FEAT), lambda b, j: (b, j, 0)),
            pl.BlockSpec((1, _PBLK, _HEADS), lambda b, j: (b, j, 0)),
            pl.BlockSpec((1, _PBLK, _HEADS), lambda b, j: (b, j, 0)),
        ],
        out_shape=[
            jax.ShapeDtypeStruct((_B, _NP, _FEAT), jnp.float32),
            jax.ShapeDtypeStruct((_B, _NP, _HEADS), jnp.float32),
            jax.ShapeDtypeStruct((_B, _NP, _HEADS), jnp.float32),
        ],
        compiler_params=pltpu.CompilerParams(
            dimension_semantics=("parallel", "parallel")),
    )(x, w, amat)


def _attn(lo, hi, pos_t, pos_nd, h, as_, ad_t, bias, wfc, bfc, fuse_fc):
    outc = _OUT_CH if fuse_fc else _FEAT
    return pl.pallas_call(
        functools.partial(_attn_body, fuse_fc=fuse_fc),
        grid_spec=pltpu.PrefetchScalarGridSpec(
            num_scalar_prefetch=2,
            grid=(_B, _NDB),
            in_specs=[
                pl.BlockSpec((1, 3, _DBLK), lambda b, j, lo, hi: (b, 0, j)),
                pl.BlockSpec((1, _NP, 3), lambda b, j, lo, hi: (b, 0, 0)),
                pl.BlockSpec((1, _NP, _FEAT), lambda b, j, lo, hi: (b, 0, 0)),
                pl.BlockSpec((1, _NP, _HEADS), lambda b, j, lo, hi: (b, 0, 0)),
                pl.BlockSpec((1, _HEADS, _DBLK), lambda b, j, lo, hi: (b, 0, j)),
                pl.BlockSpec((1, _FEAT), lambda b, j, lo, hi: (0, 0)),
                pl.BlockSpec((_FEAT, _OUT_CH), lambda b, j, lo, hi: (0, 0)),
                pl.BlockSpec((1, _OUT_CH), lambda b, j, lo, hi: (0, 0)),
            ],
            out_specs=pl.BlockSpec((1, _DBLK, outc),
                                   lambda b, j, lo, hi: (b, j, 0)),
        ),
        out_shape=jax.ShapeDtypeStruct((_B, _NP, outc), jnp.float32),
        compiler_params=pltpu.CompilerParams(
            dimension_semantics=("parallel", "arbitrary")),
    )(lo, hi, pos_t, pos_nd, h, as_, ad_t, bias, wfc, bfc)


def _chunk_ranges(pos_t):
    xp = pos_t[:, 0, :]
    cs_min = xp[:, 0::_SBLK]
    cs_max = xp[:, _SBLK - 1::_SBLK]
    xd_min = xp[:, 0::_DBLK]
    xd_max = xp[:, _DBLK - 1::_DBLK]
    lo = jax.vmap(lambda a, v: jnp.searchsorted(a, v, side='left'))(
        cs_max, xd_min - _TH).astype(jnp.int32)
    hi = (jax.vmap(lambda a, v: jnp.searchsorted(a, v, side='right'))(
        cs_min, xd_max + _TH) - 1).astype(jnp.int32)
    return lo, hi


def kernel(pos, pos_non_manifold, W1, a1_src, a1_dst, b1,
           W2, a2_src, a2_dst, b2, W_fc, b_fc):
    pos_t = jnp.concatenate([pos, pos_non_manifold], axis=2)
    perm = jnp.argsort(pos_t[:, 0, :], axis=1)
    inv = jnp.argsort(perm, axis=1)
    pos_t = jnp.take_along_axis(pos_t, perm[:, None, :], axis=2)
    pos_t = jnp.pad(pos_t, ((0, 0), (0, 0), (0, _NP - _N)),
                    constant_values=_PAD_VAL)
    pos_nd = pos_t.transpose(0, 2, 1)
    lo, hi = _chunk_ranges(pos_t)

    b1r = b1.reshape(1, _FEAT)
    b2r = b2.reshape(1, _FEAT)
    bfc = b_fc.reshape(1, _OUT_CH)

    h, as_, ad = _proj(pos_nd, W1, _amat(a1_src, a1_dst))
    x1 = _attn(lo, hi, pos_t, pos_nd, h, as_, ad.transpose(0, 2, 1),
               b1r, W_fc, bfc, fuse_fc=False)

    h2, as2, ad2 = _proj(x1, W2, _amat(a2_src, a2_dst))
    y = _attn(lo, hi, pos_t, pos_nd, h2, as2, ad2.transpose(0, 2, 1),
              b2r, W_fc, bfc, fuse_fc=True)

    y = jnp.take_along_axis(y[:, :_N], inv[:, :, None], axis=1)
    return y[:, _N_NM:_N].reshape(_B, _OUT_CH, _N_SURF)


# speedup vs baseline: 3.7951x; 1.1178x over previous
import functools

import jax
import jax.numpy as jnp
from jax.experimental import pallas as pl
from jax.experimental.pallas import tpu as pltpu

_B = 2
_N_SURF = 3000
_N_NM = 1000
_N = _N_SURF + _N_NM
_NP = 4096
_HEADS = 4
_HID = 64
_FEAT = _HEADS * _HID
_OUT_CH = 2
_PBLK = 512
_DBLK = 256
_SBLK = 256
_NDB = _NP // _DBLK
_NSB = _NP // _SBLK
_RADIUS = 0.05
_RADIUS_SQ = float(0.0025)
_TH = _RADIUS + 1e-3
_PAD_VAL = 100.0


def _proj_body(x_ref, w_ref, amat_ref, h_ref, as_ref, ad_ref):
    x = x_ref[0]
    w = w_ref[...]
    h = jnp.dot(x, w, preferred_element_type=jnp.float32)
    h_ref[0] = h
    aa = jnp.dot(h, amat_ref[...], preferred_element_type=jnp.float32)
    as_ref[0] = aa[:, :_HEADS]
    ad_ref[0] = aa[:, _HEADS:]


def _attn_body(lo_ref, hi_ref, posdt_ref, posnd_ref, h_ref, as_ref, adt_ref,
               b_ref, wfc_ref, bfc_ref, o_ref, *, fuse_fc):
    b = pl.program_id(0)
    j = pl.program_id(1)
    pos_dt = posdt_ref[0]
    sq_d = jnp.sum(pos_dt * pos_dt, axis=0, keepdims=True)
    ad_row = adt_ref[0]
    lo = lo_ref[b, j]
    hi = hi_ref[b, j]

    def chunk(k, carry):
        ls, accs = carry
        off = k * _SBLK
        pos_s = posnd_ref[0, pl.ds(off, _SBLK), :]
        sq_s = jnp.sum(pos_s * pos_s, axis=1, keepdims=True)
        dots = jnp.dot(pos_s, pos_dt, preferred_element_type=jnp.float32)
        d2 = sq_s + sq_d - 2.0 * dots
        mask = d2 < _RADIUS_SQ
        h_chunk = h_ref[0, pl.ds(off, _SBLK), :]
        as_chunk = as_ref[0, pl.ds(off, _SBLK), :]
        nl, na = [], []
        for hd in range(_HEADS):
            e = as_chunk[:, hd:hd + 1] + ad_row[hd:hd + 1, :]
            e = jnp.where(e >= 0, e, 0.2 * e)
            ex = jnp.where(mask, jnp.exp(e), 0.0)
            nl.append(ls[hd] + jnp.sum(ex, axis=0, keepdims=True))
            hc = h_chunk[:, hd * _HID:(hd + 1) * _HID]
            contrib = jax.lax.dot_general(
                hc, ex, (((0,), (0,)), ((), ())),
                preferred_element_type=jnp.float32)
            na.append(accs[hd] + contrib)
        return tuple(nl), tuple(na)

    init = (
        tuple(jnp.zeros((1, _DBLK), jnp.float32) for _ in range(_HEADS)),
        tuple(jnp.zeros((_HID, _DBLK), jnp.float32) for _ in range(_HEADS)),
    )
    ls, accs = jax.lax.fori_loop(lo, hi + 1, chunk, init)

    out = jnp.concatenate(
        [jnp.transpose(accs[hd] / (ls[hd] + 1e-16)) for hd in range(_HEADS)],
        axis=1)
    out = jnp.maximum(out + b_ref[...], 0.0)
    if fuse_fc:
        out = jnp.dot(out, wfc_ref[...],
                      preferred_element_type=jnp.float32) + bfc_ref[...]
    o_ref[0] = out


def _amat(a_src, a_dst):
    eye = jnp.eye(_HEADS, dtype=jnp.float32)
    ms = (a_src[:, :, None] * eye[:, None, :]).reshape(_FEAT, _HEADS)
    md = (a_dst[:, :, None] * eye[:, None, :]).reshape(_FEAT, _HEADS)
    return jnp.concatenate([ms, md], axis=1)


def _proj(x, w, amat):
    cin = x.shape[-1]
    return pl.pallas_call(
        _proj_body,
        grid=(_B, _NP // _PBLK),
        in_specs=[
            pl.BlockSpec((1, _PBLK, cin), lambda b, j: (b, j, 0)),
            pl.BlockSpec((cin, _FEAT), lambda b, j: (0, 0)),
            pl.BlockSpec((_FEAT, 2 * _HEADS), lambda b, j: (0, 0)),
        ],
        out_specs=[
            pl.BlockSpec((1, _PBLK, _FEAT), lambda b, j: (b, j, 0)),
            pl.BlockSpec((1, _PBLK, _HEADS), lambda b, j: (b, j, 0)),
            pl.BlockSpec((1, _PBLK, _HEADS), lambda b, j: (b, j, 0)),
        ],
        out_shape=[
            jax.ShapeDtypeStruct((_B, _NP, _FEAT), jnp.float32),
            jax.ShapeDtypeStruct((_B, _NP, _HEADS), jnp.float32),
            jax.ShapeDtypeStruct((_B, _NP, _HEADS), jnp.float32),
        ],
        compiler_params=pltpu.CompilerParams(
            dimension_semantics=("parallel", "parallel")),
    )(x, w, amat)


def _attn(lo, hi, pos_t, pos_nd, h, as_, ad_t, bias, wfc, bfc, fuse_fc):
    outc = _OUT_CH if fuse_fc else _FEAT
    return pl.pallas_call(
        functools.partial(_attn_body, fuse_fc=fuse_fc),
        grid_spec=pltpu.PrefetchScalarGridSpec(
            num_scalar_prefetch=2,
            grid=(_B, _NDB),
            in_specs=[
                pl.BlockSpec((1, 3, _DBLK), lambda b, j, lo, hi: (b, 0, j)),
                pl.BlockSpec((1, _NP, 3), lambda b, j, lo, hi: (b, 0, 0)),
                pl.BlockSpec((1, _NP, _FEAT), lambda b, j, lo, hi: (b, 0, 0)),
                pl.BlockSpec((1, _NP, _HEADS), lambda b, j, lo, hi: (b, 0, 0)),
                pl.BlockSpec((1, _HEADS, _DBLK), lambda b, j, lo, hi: (b, 0, j)),
                pl.BlockSpec((1, _FEAT), lambda b, j, lo, hi: (0, 0)),
                pl.BlockSpec((_FEAT, _OUT_CH), lambda b, j, lo, hi: (0, 0)),
                pl.BlockSpec((1, _OUT_CH), lambda b, j, lo, hi: (0, 0)),
            ],
            out_specs=pl.BlockSpec((1, _DBLK, outc),
                                   lambda b, j, lo, hi: (b, j, 0)),
        ),
        out_shape=jax.ShapeDtypeStruct((_B, _NP, outc), jnp.float32),
        compiler_params=pltpu.CompilerParams(
            dimension_semantics=("parallel", "arbitrary")),
    )(lo, hi, pos_t, pos_nd, h, as_, ad_t, bias, wfc, bfc)


def _chunk_ranges(pos_t):
    xp = pos_t[:, 0, :]
    cs_min = xp[:, 0::_SBLK]
    cs_max = xp[:, _SBLK - 1::_SBLK]
    xd_min = xp[:, 0::_DBLK]
    xd_max = xp[:, _DBLK - 1::_DBLK]
    lo = jax.vmap(lambda a, v: jnp.searchsorted(a, v, side='left'))(
        cs_max, xd_min - _TH).astype(jnp.int32)
    hi = (jax.vmap(lambda a, v: jnp.searchsorted(a, v, side='right'))(
        cs_min, xd_max + _TH) - 1).astype(jnp.int32)
    return lo, hi


def kernel(pos, pos_non_manifold, W1, a1_src, a1_dst, b1,
           W2, a2_src, a2_dst, b2, W_fc, b_fc):
    pos_t = jnp.concatenate([pos, pos_non_manifold], axis=2)
    perm = jnp.argsort(pos_t[:, 0, :], axis=1)
    inv = jnp.argsort(perm, axis=1)
    pos_t = jnp.take_along_axis(pos_t, perm[:, None, :], axis=2)
    pos_t = jnp.pad(pos_t, ((0, 0), (0, 0), (0, _NP - _N)),
                    constant_values=_PAD_VAL)
    pos_nd = pos_t.transpose(0, 2, 1)
    pos_nd0 = pos_nd.at[:, _N:, :].set(0.0)
    lo, hi = _chunk_ranges(pos_t)

    b1r = b1.reshape(1, _FEAT)
    b2r = b2.reshape(1, _FEAT)
    bfc = b_fc.reshape(1, _OUT_CH)

    h, as_, ad = _proj(pos_nd0, W1, _amat(a1_src, a1_dst))
    x1 = _attn(lo, hi, pos_t, pos_nd, h, as_, ad.transpose(0, 2, 1),
               b1r, W_fc, bfc, fuse_fc=False)

    h2, as2, ad2 = _proj(x1, W2, _amat(a2_src, a2_dst))
    y = _attn(lo, hi, pos_t, pos_nd, h2, as2, ad2.transpose(0, 2, 1),
              b2r, W_fc, bfc, fuse_fc=True)

    y = jnp.take_along_axis(y[:, :_N], inv[:, :, None], axis=1)
    return y[:, _N_NM:_N].reshape(_B, _OUT_CH, _N_SURF)


# speedup vs baseline: 3.8588x; 1.0168x over previous
import functools

import jax
import jax.numpy as jnp
from jax.experimental import pallas as pl
from jax.experimental.pallas import tpu as pltpu

_B = 2
_N_SURF = 3000
_N_NM = 1000
_N = _N_SURF + _N_NM
_NP = 4096
_HEADS = 4
_HID = 64
_FEAT = _HEADS * _HID
_OUT_CH = 2
_PBLK = 512
_DBLK = 256
_SBLK = 256
_NDB = _NP // _DBLK
_NSB = _NP // _SBLK
_RADIUS = 0.05
_RADIUS_SQ = float(0.0025)
_TH = _RADIUS + 1e-3
_PAD_VAL = 100.0


def _proj_body(x_ref, w_ref, amat_ref, h_ref, as_ref, ad_ref):
    x = x_ref[0]
    w = w_ref[...]
    h = jnp.dot(x, w, preferred_element_type=jnp.float32)
    h_ref[0] = h
    aa = jnp.dot(h, amat_ref[...], preferred_element_type=jnp.float32)
    as_ref[0] = aa[:, :_HEADS]
    ad_ref[0] = aa[:, _HEADS:]


def _attn_body(lo_ref, hi_ref, posdt_ref, posnd_ref, h_ref, as_ref, adt_ref,
               b_ref, wfc_ref, bfc_ref, *o_refs, fuse_fc):
    b = pl.program_id(0)
    j = pl.program_id(1)
    pos_dt = posdt_ref[0]
    sq_d = jnp.sum(pos_dt * pos_dt, axis=0, keepdims=True)
    ad_row = adt_ref[0]
    lo = lo_ref[b, j]
    hi = hi_ref[b, j]

    def chunk(k, carry):
        ls, accs = carry
        off = k * _SBLK
        pos_s = posnd_ref[0, pl.ds(off, _SBLK), :]
        sq_s = jnp.sum(pos_s * pos_s, axis=1, keepdims=True)
        dots = jnp.dot(pos_s, pos_dt, preferred_element_type=jnp.float32)
        d2 = sq_s + sq_d - 2.0 * dots
        mask = d2 < _RADIUS_SQ
        h_chunk = h_ref[0, pl.ds(off, _SBLK), :]
        as_chunk = as_ref[0, pl.ds(off, _SBLK), :]
        nl, na = [], []
        for hd in range(_HEADS):
            e = as_chunk[:, hd:hd + 1] + ad_row[hd:hd + 1, :]
            e = jnp.where(e >= 0, e, 0.2 * e)
            ex = jnp.where(mask, jnp.exp(e), 0.0)
            nl.append(ls[hd] + jnp.sum(ex, axis=0, keepdims=True))
            hc = h_chunk[:, hd * _HID:(hd + 1) * _HID]
            contrib = jax.lax.dot_general(
                hc, ex, (((0,), (0,)), ((), ())),
                preferred_element_type=jnp.float32)
            na.append(accs[hd] + contrib)
        return tuple(nl), tuple(na)

    init = (
        tuple(jnp.zeros((1, _DBLK), jnp.float32) for _ in range(_HEADS)),
        tuple(jnp.zeros((_HID, _DBLK), jnp.float32) for _ in range(_HEADS)),
    )
    ls, accs = jax.lax.fori_loop(lo, hi + 1, chunk, init)

    out = jnp.concatenate(
        [jnp.transpose(accs[hd] / (ls[hd] + 1e-16)) for hd in range(_HEADS)],
        axis=1)
    out = jnp.maximum(out + b_ref[...], 0.0)
    if fuse_fc:
        y = jnp.dot(out, wfc_ref[...],
                    preferred_element_type=jnp.float32) + bfc_ref[...]
        o_refs[0][0] = y
    else:
        h2 = jnp.dot(out, wfc_ref[...], preferred_element_type=jnp.float32)
        aa2 = jnp.dot(h2, bfc_ref[...], preferred_element_type=jnp.float32)
        o_refs[0][0] = h2
        o_refs[1][0] = aa2[:, :_HEADS]
        o_refs[2][0] = aa2[:, _HEADS:]


def _amat(a_src, a_dst):
    eye = jnp.eye(_HEADS, dtype=jnp.float32)
    ms = (a_src[:, :, None] * eye[:, None, :]).reshape(_FEAT, _HEADS)
    md = (a_dst[:, :, None] * eye[:, None, :]).reshape(_FEAT, _HEADS)
    return jnp.concatenate([ms, md], axis=1)


def _proj(x, w, amat):
    cin = x.shape[-1]
    return pl.pallas_call(
        _proj_body,
        grid=(_B, _NP // _PBLK),
        in_specs=[
            pl.BlockSpec((1, _PBLK, cin), lambda b, j: (b, j, 0)),
            pl.BlockSpec((cin, _FEAT), lambda b, j: (0, 0)),
            pl.BlockSpec((_FEAT, 2 * _HEADS), lambda b, j: (0, 0)),
        ],
        out_specs=[
            pl.BlockSpec((1, _PBLK, _FEAT), lambda b, j: (b, j, 0)),
            pl.BlockSpec((1, _PBLK, _HEADS), lambda b, j: (b, j, 0)),
            pl.BlockSpec((1, _PBLK, _HEADS), lambda b, j: (b, j, 0)),
        ],
        out_shape=[
            jax.ShapeDtypeStruct((_B, _NP, _FEAT), jnp.float32),
            jax.ShapeDtypeStruct((_B, _NP, _HEADS), jnp.float32),
            jax.ShapeDtypeStruct((_B, _NP, _HEADS), jnp.float32),
        ],
        compiler_params=pltpu.CompilerParams(
            dimension_semantics=("parallel", "parallel")),
    )(x, w, amat)


def _attn(lo, hi, pos_t, pos_nd, h, as_, ad_t, bias, wfc, bfc, fuse_fc):
    if fuse_fc:
        wshape = (_FEAT, _OUT_CH)
        bshape = (1, _OUT_CH)
        out_specs = pl.BlockSpec((1, _DBLK, _OUT_CH),
                                 lambda b, j, lo, hi: (b, j, 0))
        out_shape = jax.ShapeDtypeStruct((_B, _NP, _OUT_CH), jnp.float32)
    else:
        wshape = (_FEAT, _FEAT)
        bshape = (_FEAT, 2 * _HEADS)
        out_specs = [
            pl.BlockSpec((1, _DBLK, _FEAT), lambda b, j, lo, hi: (b, j, 0)),
            pl.BlockSpec((1, _DBLK, _HEADS), lambda b, j, lo, hi: (b, j, 0)),
            pl.BlockSpec((1, _DBLK, _HEADS), lambda b, j, lo, hi: (b, j, 0)),
        ]
        out_shape = [
            jax.ShapeDtypeStruct((_B, _NP, _FEAT), jnp.float32),
            jax.ShapeDtypeStruct((_B, _NP, _HEADS), jnp.float32),
            jax.ShapeDtypeStruct((_B, _NP, _HEADS), jnp.float32),
        ]
    return pl.pallas_call(
        functools.partial(_attn_body, fuse_fc=fuse_fc),
        grid_spec=pltpu.PrefetchScalarGridSpec(
            num_scalar_prefetch=2,
            grid=(_B, _NDB),
            in_specs=[
                pl.BlockSpec((1, 3, _DBLK), lambda b, j, lo, hi: (b, 0, j)),
                pl.BlockSpec((1, _NP, 3), lambda b, j, lo, hi: (b, 0, 0)),
                pl.BlockSpec((1, _NP, _FEAT), lambda b, j, lo, hi: (b, 0, 0)),
                pl.BlockSpec((1, _NP, _HEADS), lambda b, j, lo, hi: (b, 0, 0)),
                pl.BlockSpec((1, _HEADS, _DBLK), lambda b, j, lo, hi: (b, 0, j)),
                pl.BlockSpec((1, _FEAT), lambda b, j, lo, hi: (0, 0)),
                pl.BlockSpec(wshape, lambda b, j, lo, hi: (0, 0)),
                pl.BlockSpec(bshape, lambda b, j, lo, hi: (0, 0)),
            ],
            out_specs=out_specs,
        ),
        out_shape=out_shape,
        compiler_params=pltpu.CompilerParams(
            dimension_semantics=("parallel", "arbitrary")),
    )(lo, hi, pos_t, pos_nd, h, as_, ad_t, bias, wfc, bfc)


def _chunk_ranges(pos_t):
    xp = pos_t[:, 0, :]
    cs_min = xp[:, 0::_SBLK]
    cs_max = xp[:, _SBLK - 1::_SBLK]
    xd_min = xp[:, 0::_DBLK]
    xd_max = xp[:, _DBLK - 1::_DBLK]
    lo = jax.vmap(lambda a, v: jnp.searchsorted(a, v, side='left'))(
        cs_max, xd_min - _TH).astype(jnp.int32)
    hi = (jax.vmap(lambda a, v: jnp.searchsorted(a, v, side='right'))(
        cs_min, xd_max + _TH) - 1).astype(jnp.int32)
    return lo, hi


def kernel(pos, pos_non_manifold, W1, a1_src, a1_dst, b1,
           W2, a2_src, a2_dst, b2, W_fc, b_fc):
    pos_t = jnp.concatenate([pos, pos_non_manifold], axis=2)
    perm = jnp.argsort(pos_t[:, 0, :], axis=1)
    inv = jnp.argsort(perm, axis=1)
    pos_t = jnp.take_along_axis(pos_t, perm[:, None, :], axis=2)
    pos_t = jnp.pad(pos_t, ((0, 0), (0, 0), (0, _NP - _N)),
                    constant_values=_PAD_VAL)
    pos_nd = pos_t.transpose(0, 2, 1)
    pos_nd0 = pos_nd.at[:, _N:, :].set(0.0)
    lo, hi = _chunk_ranges(pos_t)

    b1r = b1.reshape(1, _FEAT)
    b2r = b2.reshape(1, _FEAT)
    bfc = b_fc.reshape(1, _OUT_CH)

    h, as_, ad = _proj(pos_nd0, W1, _amat(a1_src, a1_dst))
    h2, as2, ad2 = _attn(lo, hi, pos_t, pos_nd, h, as_,
                         ad.transpose(0, 2, 1), b1r, W2,
                         _amat(a2_src, a2_dst), fuse_fc=False)
    y = _attn(lo, hi, pos_t, pos_nd, h2, as2, ad2.transpose(0, 2, 1),
              b2r, W_fc, bfc, fuse_fc=True)

    y = jnp.take_along_axis(y[:, :_N], inv[:, :, None], axis=1)
    return y[:, _N_NM:_N].reshape(_B, _OUT_CH, _N_SURF)
